# baseline (speedup 1.0000x reference)
import functools

import jax
import jax.numpy as jnp
from jax import lax
from jax.experimental import pallas as pl
from jax.experimental.pallas import tpu as pltpu
from jax.experimental.pallas import tpu_sc as plsc

_NC = 2
_NS = 16
_L = 16
_NW = _NC * _NS


def _make_sc_kernel(P, H, W, C, N):
    PW = P // _NW
    n_chunks = PW // N
    n_groups = N // _L
    assert n_chunks % 2 == 0

    mesh = plsc.VectorSubcoreMesh(
        core_axis_name="c", subcore_axis_name="s",
        num_cores=_NC, num_subcores=_NS)

    wf = jnp.float32(W)
    hf = jnp.float32(H)

    def axis_terms(v, extent):
        ip = ((v + 1.0) * extent - 1.0) * 0.5
        t0 = ip.astype(jnp.int32)
        t0 = jnp.where(t0.astype(jnp.float32) > ip, t0 - 1, t0)
        f0 = t0.astype(jnp.float32)
        w1 = ip - f0
        w0 = 1.0 - w1
        t1 = t0 + 1
        lim = extent - 1.0
        in0 = (f0 >= 0.0) & (f0 <= lim)
        in1 = (f0 + 1.0 >= 0.0) & (f0 + 1.0 <= lim)
        w0 = jnp.where(in0, w0, 0.0)
        w1 = jnp.where(in1, w1, 0.0)
        ilim = jnp.int32(extent) - 1
        c0 = jnp.minimum(jnp.maximum(t0, 0), ilim)
        c1 = jnp.minimum(jnp.maximum(t1, 0), ilim)
        return c0, c1, w0, w1

    def body(pts_hbm, table_hbm, out_hbm, *refs):
        it = iter(refs)
        pts_v = [next(it) for _ in range(2)]
        idx_v = [[next(it) for _ in range(4)] for _ in range(2)]
        w_v = [[next(it) for _ in range(4)] for _ in range(2)]
        rows_v = [[next(it) for _ in range(4)] for _ in range(2)]
        out_v = [next(it) for _ in range(2)]
        tsc_v = next(it)
        psem = [next(it) for _ in range(2)]
        gsem = [next(it) for _ in range(2)]
        osem = [next(it) for _ in range(2)]

        wid = lax.axis_index("s") * _NC + lax.axis_index("c")
        base0 = wid * PW

        def pts_fire(t, s):
            pltpu.async_copy(
                pts_hbm.at[pl.ds((base0 + t * N) * 2, 2 * N)], pts_v[s], psem[s])

        def pts_wait(s):
            pltpu.make_async_copy(
                pts_hbm.at[pl.ds(0, 2 * N)], pts_v[s], psem[s]).wait()

        def compute_idx(s):
            def group_a(g, carry):
                off = g * _L
                xv = pts_v[s][pl.ds(off, _L)]
                yv = pts_v[s][pl.ds(N + off, _L)]
                x0, x1, wx0, wx1 = axis_terms(xv, wf)
                y0, y1, wy0, wy1 = axis_terms(yv, hf)
                r0 = y0 * W
                r1 = y1 * W
                idx_v[s][0][pl.ds(off, _L)] = r0 + x0
                idx_v[s][1][pl.ds(off, _L)] = r0 + x1
                idx_v[s][2][pl.ds(off, _L)] = r1 + x0
                idx_v[s][3][pl.ds(off, _L)] = r1 + x1
                w_v[s][0][pl.ds(off, _L)] = wx0 * wy0
                w_v[s][1][pl.ds(off, _L)] = wx1 * wy0
                w_v[s][2][pl.ds(off, _L)] = wx0 * wy1
                w_v[s][3][pl.ds(off, _L)] = wx1 * wy1
                return carry
            lax.fori_loop(0, n_groups, group_a, 0)

        def gathers_fire(s):
            for k in range(4):
                pltpu.async_copy(table_hbm.at[idx_v[s][k]], rows_v[s][k], gsem[s])

        def gathers_wait(s):
            for k in range(4):
                pltpu.make_async_copy(
                    table_hbm.at[idx_v[s][k]], rows_v[s][k], gsem[s]).wait()

        lane = lax.iota(jnp.int32, _L)
        lane17 = lane * 17

        def blend(s):
            def group_b(g, carry):
                off = g * _L
                w00g = w_v[s][0][pl.ds(off, _L)]
                w01g = w_v[s][1][pl.ds(off, _L)]
                w10g = w_v[s][2][pl.ds(off, _L)]
                w11g = w_v[s][3][pl.ds(off, _L)]
                vals_lo = []
                vals_hi = []
                for j in range(_L):
                    p = off + j
                    w00 = w00g[j]
                    w01 = w01g[j]
                    w10 = w10g[j]
                    w11 = w11g[j]
                    sl = pl.ds(0, _L)
                    lo = []
                    hi = []
                    for k in range(4):
                        r = plsc.bitcast(rows_v[s][k][p, sl], jnp.bfloat16)
                        a, b = plsc.unpack(
                            r, format=plsc.PackFormat.INTERLEAVED,
                            preferred_element_type=jnp.float32)
                        lo.append(a)
                        hi.append(b)
                    vals_lo.append(w00 * lo[0] + w01 * lo[1]
                                   + w10 * lo[2] + w11 * lo[3])
                    vals_hi.append(w00 * hi[0] + w01 * hi[1]
                                   + w10 * hi[2] + w11 * hi[3])
                for h, vals in ((0, vals_lo), (_L, vals_hi)):
                    for j in range(_L):
                        plsc.store_scatter(tsc_v, [lane17 + j], vals[j])
                    for i in range(_L):
                        c = h + i
                        tile_off = (c >> 3) * 1024 + (c & 7) * 128
                        col = plsc.load_gather(tsc_v, [lane + 17 * i])
                        out_v[s][pl.ds(tile_off + off, _L)] = col
                return carry
            lax.fori_loop(0, n_groups, group_b, 0)

        n_pblocks = P // N

        def out_fire(t, s):
            pb = wid * n_chunks + t
            for cb in range(C // 8):
                pltpu.async_copy(
                    out_v[s].at[pl.ds(cb * 1024, 1024)],
                    out_hbm.at[pl.ds((cb * n_pblocks + pb) * 1024, 1024)],
                    osem[s])

        def out_wait(s):
            for _ in range(C // 8):
                pltpu.make_async_copy(
                    out_v[s].at[pl.ds(0, 1024)],
                    out_hbm.at[pl.ds(0, 1024)], osem[s]).wait()

        pts_fire(0, 0)
        pts_fire(1, 1)
        pts_wait(0)
        compute_idx(0)
        gathers_fire(0)

        def pair_body(q, carry):
            for par in (0, 1):
                t = 2 * q + par
                nxt = 1 - par

                @pl.when(t + 2 < n_chunks)
                def _():
                    pts_fire(t + 2, par)

                @pl.when(t + 1 < n_chunks)
                def _():
                    pts_wait(nxt)
                    compute_idx(nxt)
                    gathers_fire(nxt)

                gathers_wait(par)

                @pl.when(t >= 2)
                def _():
                    out_wait(par)

                blend(par)
                out_fire(t, par)
            return carry

        lax.fori_loop(0, n_chunks // 2, pair_body, 0)

        out_wait(0)
        out_wait(1)

    scratch = (
        [pltpu.VMEM((2 * N,), jnp.float32) for _ in range(2)]
        + [pltpu.VMEM((N,), jnp.int32) for _ in range(8)]
        + [pltpu.VMEM((N,), jnp.float32) for _ in range(8)]
        + [pltpu.VMEM((N, C // 2), jnp.int32) for _ in range(8)]
        + [pltpu.VMEM(((C // 8) * 1024,), jnp.float32) for _ in range(2)]
        + [pltpu.VMEM((_L * 17,), jnp.float32)]
        + [pltpu.SemaphoreType.DMA for _ in range(6)]
    )

    return pl.kernel(
        body,
        out_type=jax.ShapeDtypeStruct((P * C,), jnp.float32),
        mesh=mesh,
        compiler_params=pltpu.CompilerParams(
            use_tc_tiling_on_sc=False, needs_layout_passes=False),
        scratch_types=scratch,
    )


def _make_table_prep(C, H, W):
    def body(in_ref, out_ref):
        x = in_ref[...]
        lo = jax.lax.bitcast_convert_type(
            x[:C // 2].astype(jnp.bfloat16), jnp.uint16).astype(jnp.uint32)
        hi = jax.lax.bitcast_convert_type(
            x[C // 2:].astype(jnp.bfloat16), jnp.uint16).astype(jnp.uint32)
        w = ((hi << 16) | lo).astype(jnp.int32)
        nw = C // 2
        for a in range(8):
            ta = jnp.transpose(w[:, a, :])
            ta3 = ta.reshape(W * nw // 128, 128 // nw, nw)
            merged = jnp.concatenate(
                [ta3[:, b, :] for b in range(128 // nw)], axis=1)
            out_ref[pl.ds(a * (W * nw // 128), W * nw // 128), :] = merged

    rows_per_blk = 8 * W * (C // 2) // 128
    return pl.pallas_call(
        body,
        grid=(H // 8,),
        in_specs=[pl.BlockSpec((C, 8, W), lambda i: (0, i, 0))],
        out_specs=pl.BlockSpec((rows_per_blk, 128), lambda i: (i, 0)),
        out_shape=jax.ShapeDtypeStruct((H * W * (C // 2) // 128, 128),
                                       jnp.int32),
    )


@functools.partial(jax.jit, static_argnames=())
def kernel(x, plane):
    C = plane.shape[1]
    H = plane.shape[2]
    W = plane.shape[3]
    pts = x.reshape(-1, 2)
    P = pts.shape[0]
    prep = _make_table_prep(C, H, W)
    table = prep(plane.reshape(C, H, W)).reshape(H * W, C // 2)
    pts_lin = pts.reshape(P // 128, 128, 2).transpose(0, 2, 1).reshape(2 * P)
    sc = _make_sc_kernel(P, H, W, C, 128)
    out = sc(pts_lin, table)
    a4 = out.reshape(C // 8, P // 128, 8, 128)
    out2 = a4.transpose(1, 3, 0, 2).reshape(P, C)
    return out2.reshape(x.shape[:-1] + (C,))


# speedup vs baseline: 1.6063x; 1.6063x over previous
import functools

import jax
import jax.numpy as jnp
from jax import lax
from jax.experimental import pallas as pl
from jax.experimental.pallas import tpu as pltpu
from jax.experimental.pallas import tpu_sc as plsc

_NC = 2
_NS = 16
_L = 16
_NW = _NC * _NS


def _make_sc_kernel(P, H, W, C, N):
    PW = P // _NW
    n_chunks = PW // N
    n_groups = N // _L
    assert n_chunks % 2 == 0

    mesh = plsc.VectorSubcoreMesh(
        core_axis_name="c", subcore_axis_name="s",
        num_cores=_NC, num_subcores=_NS)

    wf = jnp.float32(W)
    hf = jnp.float32(H)

    def axis_terms(v, extent):
        ip = ((v + 1.0) * extent - 1.0) * 0.5
        t0 = ip.astype(jnp.int32)
        t0 = jnp.where(t0.astype(jnp.float32) > ip, t0 - 1, t0)
        f0 = t0.astype(jnp.float32)
        w1 = ip - f0
        w0 = 1.0 - w1
        t1 = t0 + 1
        lim = extent - 1.0
        in0 = (f0 >= 0.0) & (f0 <= lim)
        in1 = (f0 + 1.0 >= 0.0) & (f0 + 1.0 <= lim)
        w0 = jnp.where(in0, w0, 0.0)
        w1 = jnp.where(in1, w1, 0.0)
        ilim = jnp.int32(extent) - 1
        c0 = jnp.minimum(jnp.maximum(t0, 0), ilim)
        c1 = jnp.minimum(jnp.maximum(t1, 0), ilim)
        return c0, c1, w0, w1

    def body(pts_hbm, table_hbm, out_hbm, *refs):
        it = iter(refs)
        pts_v = [next(it) for _ in range(2)]
        idx_v = [[next(it) for _ in range(4)] for _ in range(2)]
        w_v = [[next(it) for _ in range(4)] for _ in range(2)]
        rows_v = [[next(it) for _ in range(4)] for _ in range(2)]
        out_v = [next(it) for _ in range(2)]
        psem = [next(it) for _ in range(2)]
        gsem = [next(it) for _ in range(2)]
        osem = [next(it) for _ in range(2)]

        wid = lax.axis_index("s") * _NC + lax.axis_index("c")
        base0 = wid * PW

        def pts_fire(t, s):
            pltpu.async_copy(
                pts_hbm.at[pl.ds((base0 + t * N) * 2, 2 * N)], pts_v[s], psem[s])

        def pts_wait(s):
            pltpu.make_async_copy(
                pts_hbm.at[pl.ds(0, 2 * N)], pts_v[s], psem[s]).wait()

        def compute_idx(s):
            def group_a(g, carry):
                off = g * _L
                xv = pts_v[s][pl.ds(off, _L)]
                yv = pts_v[s][pl.ds(N + off, _L)]
                x0, x1, wx0, wx1 = axis_terms(xv, wf)
                y0, y1, wy0, wy1 = axis_terms(yv, hf)
                r0 = y0 * W
                r1 = y1 * W
                idx_v[s][0][pl.ds(off, _L)] = r0 + x0
                idx_v[s][1][pl.ds(off, _L)] = r0 + x1
                idx_v[s][2][pl.ds(off, _L)] = r1 + x0
                idx_v[s][3][pl.ds(off, _L)] = r1 + x1
                w_v[s][0][pl.ds(off, _L)] = wx0 * wy0
                w_v[s][1][pl.ds(off, _L)] = wx1 * wy0
                w_v[s][2][pl.ds(off, _L)] = wx0 * wy1
                w_v[s][3][pl.ds(off, _L)] = wx1 * wy1
                return carry
            lax.fori_loop(0, n_groups, group_a, 0)

        def gathers_fire(s):
            for k in range(4):
                pltpu.async_copy(table_hbm.at[idx_v[s][k]], rows_v[s][k], gsem[s])

        def gathers_wait(s):
            for k in range(4):
                pltpu.make_async_copy(
                    table_hbm.at[idx_v[s][k]], rows_v[s][k], gsem[s]).wait()

        lane = lax.iota(jnp.int32, _L)
        perm_idx = {k: lane ^ k for k in (1, 2, 4, 8)}
        sel_mask = {k: (lane & k) == 0 for k in (1, 2, 4, 8)}

        def _perm(v, k):
            return v.at[perm_idx[k]].get(mode="promise_in_bounds")

        def transpose16(m):
            for k in (8, 4, 2, 1):
                nm = list(m)
                for i in range(_L):
                    if i & k:
                        continue
                    j = i | k
                    a, b = m[i], m[j]
                    pa, pb = _perm(a, k), _perm(b, k)
                    nm[i] = jnp.where(sel_mask[k], a, pb)
                    nm[j] = jnp.where(sel_mask[k], pa, b)
                m = nm
            return m

        def blend(s):
            def group_b(g, carry):
                off = g * _L
                w00g = w_v[s][0][pl.ds(off, _L)]
                w01g = w_v[s][1][pl.ds(off, _L)]
                w10g = w_v[s][2][pl.ds(off, _L)]
                w11g = w_v[s][3][pl.ds(off, _L)]
                vals_lo = []
                vals_hi = []
                for j in range(_L):
                    p = off + j
                    w00 = w00g[j]
                    w01 = w01g[j]
                    w10 = w10g[j]
                    w11 = w11g[j]
                    sl = pl.ds(0, _L)
                    lo = []
                    hi = []
                    for k in range(4):
                        r = plsc.bitcast(rows_v[s][k][p, sl], jnp.bfloat16)
                        a, b = plsc.unpack(
                            r, format=plsc.PackFormat.INTERLEAVED,
                            preferred_element_type=jnp.float32)
                        lo.append(a)
                        hi.append(b)
                    vals_lo.append(w00 * lo[0] + w01 * lo[1]
                                   + w10 * lo[2] + w11 * lo[3])
                    vals_hi.append(w00 * hi[0] + w01 * hi[1]
                                   + w10 * hi[2] + w11 * hi[3])
                for h, vals in ((0, vals_lo), (_L, vals_hi)):
                    cols = transpose16(vals)
                    for i in range(_L):
                        c = h + i
                        tile_off = (c >> 3) * 1024 + (c & 7) * 128
                        out_v[s][pl.ds(tile_off + off, _L)] = cols[i]
                return carry
            lax.fori_loop(0, n_groups, group_b, 0)

        n_pblocks = P // N

        def out_fire(t, s):
            pb = wid * n_chunks + t
            for cb in range(C // 8):
                pltpu.async_copy(
                    out_v[s].at[pl.ds(cb * 1024, 1024)],
                    out_hbm.at[pl.ds((cb * n_pblocks + pb) * 1024, 1024)],
                    osem[s])

        def out_wait(s):
            for _ in range(C // 8):
                pltpu.make_async_copy(
                    out_v[s].at[pl.ds(0, 1024)],
                    out_hbm.at[pl.ds(0, 1024)], osem[s]).wait()

        pts_fire(0, 0)
        pts_fire(1, 1)
        pts_wait(0)
        compute_idx(0)
        gathers_fire(0)

        def pair_body(q, carry):
            for par in (0, 1):
                t = 2 * q + par
                nxt = 1 - par

                @pl.when(t + 2 < n_chunks)
                def _():
                    pts_fire(t + 2, par)

                @pl.when(t + 1 < n_chunks)
                def _():
                    pts_wait(nxt)
                    compute_idx(nxt)
                    gathers_fire(nxt)

                gathers_wait(par)

                @pl.when(t >= 2)
                def _():
                    out_wait(par)

                blend(par)
                out_fire(t, par)
            return carry

        lax.fori_loop(0, n_chunks // 2, pair_body, 0)

        out_wait(0)
        out_wait(1)

    scratch = (
        [pltpu.VMEM((2 * N,), jnp.float32) for _ in range(2)]
        + [pltpu.VMEM((N,), jnp.int32) for _ in range(8)]
        + [pltpu.VMEM((N,), jnp.float32) for _ in range(8)]
        + [pltpu.VMEM((N, C // 2), jnp.int32) for _ in range(8)]
        + [pltpu.VMEM(((C // 8) * 1024,), jnp.float32) for _ in range(2)]
        + [pltpu.SemaphoreType.DMA for _ in range(6)]
    )

    return pl.kernel(
        body,
        out_type=jax.ShapeDtypeStruct((P * C,), jnp.float32),
        mesh=mesh,
        compiler_params=pltpu.CompilerParams(
            use_tc_tiling_on_sc=False, needs_layout_passes=False),
        scratch_types=scratch,
    )


def _make_table_prep(C, H, W):
    def body(in_ref, out_ref):
        x = in_ref[...]
        lo = jax.lax.bitcast_convert_type(
            x[:C // 2].astype(jnp.bfloat16), jnp.uint16).astype(jnp.uint32)
        hi = jax.lax.bitcast_convert_type(
            x[C // 2:].astype(jnp.bfloat16), jnp.uint16).astype(jnp.uint32)
        w = ((hi << 16) | lo).astype(jnp.int32)
        nw = C // 2
        for a in range(8):
            ta = jnp.transpose(w[:, a, :])
            ta3 = ta.reshape(W * nw // 128, 128 // nw, nw)
            merged = jnp.concatenate(
                [ta3[:, b, :] for b in range(128 // nw)], axis=1)
            out_ref[pl.ds(a * (W * nw // 128), W * nw // 128), :] = merged

    rows_per_blk = 8 * W * (C // 2) // 128
    return pl.pallas_call(
        body,
        grid=(H // 8,),
        in_specs=[pl.BlockSpec((C, 8, W), lambda i: (0, i, 0))],
        out_specs=pl.BlockSpec((rows_per_blk, 128), lambda i: (i, 0)),
        out_shape=jax.ShapeDtypeStruct((H * W * (C // 2) // 128, 128),
                                       jnp.int32),
    )


@functools.partial(jax.jit, static_argnames=())
def kernel(x, plane):
    C = plane.shape[1]
    H = plane.shape[2]
    W = plane.shape[3]
    pts = x.reshape(-1, 2)
    P = pts.shape[0]
    prep = _make_table_prep(C, H, W)
    table = prep(plane.reshape(C, H, W)).reshape(H * W, C // 2)
    pts_lin = pts.reshape(P // 128, 128, 2).transpose(0, 2, 1).reshape(2 * P)
    sc = _make_sc_kernel(P, H, W, C, 128)
    out = sc(pts_lin, table)
    a4 = out.reshape(C // 8, P // 128, 8, 128)
    out2 = a4.transpose(1, 3, 0, 2).reshape(P, C)
    return out2.reshape(x.shape[:-1] + (C,))
